# Initial kernel scaffold; baseline (speedup 1.0000x reference)
import jax
import jax.numpy as jnp
from jax.experimental import pallas as pl


def kernel(coordinates, edge_src, edge_dst):
    raise NotImplementedError("write your pallas kernel here")


# speedup vs baseline: 11.8970x; 11.8970x over previous
import jax
import jax.numpy as jnp
from jax import lax
from jax.experimental import pallas as pl
from jax.experimental.pallas import tpu as pltpu
from jax.experimental.pallas import tpu_sc as plsc

N_EDGES = 6_400_000
CUTOFF = 5.0

_D = 16
_NC = 2
_NS = 16
_NW = _NC * _NS
_EW = N_EDGES // _NW
_C = 2000
_NCH = _EW // _C
_G = _C // 16
_SUB = 80
_NSUB = _C // _SUB

_P0 = 0.99999528
_P1 = -0.49993091
_P2 = 0.04151172
_P3 = -0.00127871
_HALF_SCALE = 0.3141592653589793
_HMAX = 1.5707963267948966


def _sc_body(coords, src_h, dst_h, vec_o, d_o, sw_o,
             idx_s, idx_d, rows_s, rows_d, vecb, db, swb, sem_a, sem_b):
    wid = lax.axis_index("s") * _NC + lax.axis_index("c")
    iota = lax.iota(jnp.int32, 16)
    col0 = jnp.zeros((16,), jnp.int32)
    col1 = col0 + 1
    col2 = col0 + 2

    def chunk(i, carry):
        base = wid * _EW + i * _C
        pltpu.sync_copy(src_h.at[pl.ds(base, _C)], idx_s)
        pltpu.sync_copy(dst_h.at[pl.ds(base, _C)], idx_d)
        copies = []
        for j in range(_NSUB):
            sl = pl.ds(j * _SUB, _SUB)
            copies.append(pltpu.async_copy(coords.at[idx_s.at[sl]], rows_s.at[sl], sem_a))
            copies.append(pltpu.async_copy(coords.at[idx_d.at[sl]], rows_d.at[sl], sem_b))
        for cp in copies:
            cp.wait()

        def group(g, carry2):
            r = g * 16 + iota
            dx = plsc.load_gather(rows_d, [r, col0]) - plsc.load_gather(rows_s, [r, col0])
            dy = plsc.load_gather(rows_d, [r, col1]) - plsc.load_gather(rows_s, [r, col1])
            dz = plsc.load_gather(rows_d, [r, col2]) - plsc.load_gather(rows_s, [r, col2])
            plsc.store_scatter(vecb, [r, col0], dx)
            plsc.store_scatter(vecb, [r, col1], dy)
            plsc.store_scatter(vecb, [r, col2], dz)
            d2 = dx * dx + dy * dy + dz * dz
            bits = plsc.bitcast(d2, jnp.int32)
            y = plsc.bitcast(jnp.int32(0x5F3759DF) - (bits >> 1), jnp.float32)
            y = y * (1.5 - (0.5 * d2 * y) * y)
            y = y * (1.5 - (0.5 * d2 * y) * y)
            y = y * (1.5 - (0.5 * d2 * y) * y)
            d = d2 * y
            d = jnp.where(d2 == 0.0, 0.0, d)
            db[pl.ds(g * 16, 16)] = d
            h = jnp.minimum(d * _HALF_SCALE, _HMAX)
            u = h * h
            p = _P0 + u * (_P1 + u * (_P2 + u * _P3))
            swb[pl.ds(g * 16, 16)] = jnp.where(d < CUTOFF, p * p, 0.0)
            return carry2

        lax.fori_loop(0, _G, group, 0)
        pltpu.sync_copy(vecb, vec_o.at[pl.ds(base, _C)])
        pltpu.sync_copy(db, d_o.at[pl.ds(base, _C)])
        pltpu.sync_copy(swb, sw_o.at[pl.ds(base, _C)])
        return carry

    lax.fori_loop(0, _NCH, chunk, 0)


_sc_call = pl.kernel(
    _sc_body,
    out_type=[
        jax.ShapeDtypeStruct((N_EDGES, 3), jnp.float32),
        jax.ShapeDtypeStruct((N_EDGES,), jnp.float32),
        jax.ShapeDtypeStruct((N_EDGES,), jnp.float32),
    ],
    mesh=plsc.VectorSubcoreMesh(core_axis_name="c", subcore_axis_name="s"),
    compiler_params=pltpu.CompilerParams(use_tc_tiling_on_sc=False,
                                         needs_layout_passes=False),
    scratch_types=[
        pltpu.VMEM((_C,), jnp.int32),
        pltpu.VMEM((_C,), jnp.int32),
        pltpu.VMEM((_C, _D), jnp.float32),
        pltpu.VMEM((_C, _D), jnp.float32),
        pltpu.VMEM((_C, 3), jnp.float32),
        pltpu.VMEM((_C,), jnp.float32),
        pltpu.VMEM((_C,), jnp.float32),
        pltpu.SemaphoreType.DMA,
        pltpu.SemaphoreType.DMA,
    ],
)


@jax.jit
def kernel(coordinates, edge_src, edge_dst):
    table = jnp.pad(coordinates, ((0, 0), (0, _D - 3)))
    vec, distances, switch = _sc_call(table, edge_src, edge_dst)
    edge_mask = distances < CUTOFF
    return vec, distances, switch, edge_mask


# speedup vs baseline: 37.2588x; 3.1318x over previous
import jax
import jax.numpy as jnp
from jax import lax
from jax.experimental import pallas as pl
from jax.experimental.pallas import tpu as pltpu
from jax.experimental.pallas import tpu_sc as plsc

N_EDGES = 6_400_000
CUTOFF = 5.0

_D = 16
_NC = 2
_NS = 16
_NW = _NC * _NS
_C = 2048
_NB = _C // 128
_NCHUNKS = N_EDGES // _C
_FULL = _NCHUNKS // _NW
_REM = _NCHUNKS - _FULL * _NW
_G = _C // 16
_SUB = 128
_NSUB = _C // _SUB

_P0 = 0.99999528
_P1 = -0.49993091
_P2 = 0.04151172
_P3 = -0.00127871
_HALF_SCALE = 0.3141592653589793
_HMAX = 1.5707963267948966


def _sc_body(coords, src_h, dst_h, vec_o, d_o, sw_o,
             idx_s, idx_d, rows_s, rows_d, vecb, db, swb, sem_a, sem_b):
    wid = lax.axis_index("s") * _NC + lax.axis_index("c")
    iota = lax.iota(jnp.int32, 16)
    col0 = jnp.zeros((16,), jnp.int32)
    col1 = col0 + 1
    col2 = col0 + 2
    zero16 = jnp.zeros((16,), jnp.float32)

    def zinit(k, carry):
        vecb[k // 8, 3, pl.ds((k % 8) * 16, 16)] = zero16
        return carry

    lax.fori_loop(0, _NB * 8, zinit, 0)

    def chunk_body(c):
        base = c * _C
        pltpu.sync_copy(src_h.at[pl.ds(base, _C)], idx_s)
        pltpu.sync_copy(dst_h.at[pl.ds(base, _C)], idx_d)
        copies = []
        for j in range(_NSUB):
            sl = pl.ds(j * _SUB, _SUB)
            copies.append(pltpu.async_copy(coords.at[idx_s.at[sl]], rows_s.at[sl], sem_a))
            copies.append(pltpu.async_copy(coords.at[idx_d.at[sl]], rows_d.at[sl], sem_b))
        for cp in copies:
            cp.wait()

        def group(g, carry2):
            r = g * 16 + iota
            blk = g // 8
            off = (g % 8) * 16
            dx = plsc.load_gather(rows_d, [r, col0]) - plsc.load_gather(rows_s, [r, col0])
            dy = plsc.load_gather(rows_d, [r, col1]) - plsc.load_gather(rows_s, [r, col1])
            dz = plsc.load_gather(rows_d, [r, col2]) - plsc.load_gather(rows_s, [r, col2])
            vecb[blk, 0, pl.ds(off, 16)] = dx
            vecb[blk, 1, pl.ds(off, 16)] = dy
            vecb[blk, 2, pl.ds(off, 16)] = dz
            d2 = dx * dx + dy * dy + dz * dz
            bits = plsc.bitcast(d2, jnp.int32)
            y = plsc.bitcast(jnp.int32(0x5F3759DF) - (bits >> 1), jnp.float32)
            y = y * (1.5 - (0.5 * d2 * y) * y)
            y = y * (1.5 - (0.5 * d2 * y) * y)
            y = y * (1.5 - (0.5 * d2 * y) * y)
            d = d2 * y
            d = jnp.where(d2 == 0.0, 0.0, d)
            db[pl.ds(g * 16, 16)] = d
            h = jnp.minimum(d * _HALF_SCALE, _HMAX)
            u = h * h
            p = _P0 + u * (_P1 + u * (_P2 + u * _P3))
            swb[pl.ds(g * 16, 16)] = jnp.where(d < CUTOFF, p * p, 0.0)
            return carry2

        lax.fori_loop(0, _G, group, 0)
        pltpu.sync_copy(vecb, vec_o.at[pl.ds(c * _NB, _NB)])
        pltpu.sync_copy(db, d_o.at[pl.ds(base, _C)])
        pltpu.sync_copy(swb, sw_o.at[pl.ds(base, _C)])

    def round_(k, carry):
        chunk_body(wid + k * _NW)
        return carry

    lax.fori_loop(0, _FULL, round_, 0)

    @pl.when(wid < _REM)
    def _():
        chunk_body(wid + _FULL * _NW)


_sc_call = pl.kernel(
    _sc_body,
    out_type=[
        jax.ShapeDtypeStruct((N_EDGES // 128, 4, 128), jnp.float32),
        jax.ShapeDtypeStruct((N_EDGES,), jnp.float32),
        jax.ShapeDtypeStruct((N_EDGES,), jnp.float32),
    ],
    mesh=plsc.VectorSubcoreMesh(core_axis_name="c", subcore_axis_name="s"),
    compiler_params=pltpu.CompilerParams(use_tc_tiling_on_sc=False,
                                         needs_layout_passes=False),
    scratch_types=[
        pltpu.VMEM((_C,), jnp.int32),
        pltpu.VMEM((_C,), jnp.int32),
        pltpu.VMEM((_C, _D), jnp.float32),
        pltpu.VMEM((_C, _D), jnp.float32),
        pltpu.VMEM((_NB, 4, 128), jnp.float32),
        pltpu.VMEM((_C,), jnp.float32),
        pltpu.VMEM((_C,), jnp.float32),
        pltpu.SemaphoreType.DMA,
        pltpu.SemaphoreType.DMA,
    ],
)


@jax.jit
def kernel(coordinates, edge_src, edge_dst):
    table = jnp.pad(coordinates, ((0, 0), (0, _D - 3)))
    vec4, distances, switch = _sc_call(table, edge_src, edge_dst)
    vec = vec4.transpose(0, 2, 1).reshape(N_EDGES, 4)[:, :3]
    edge_mask = distances < CUTOFF
    return vec, distances, switch, edge_mask


# speedup vs baseline: 45.5420x; 1.2223x over previous
import jax
import jax.numpy as jnp
from jax import lax
from jax.experimental import pallas as pl
from jax.experimental.pallas import tpu as pltpu
from jax.experimental.pallas import tpu_sc as plsc

N_EDGES = 6_400_000
CUTOFF = 5.0

_D = 16
_NC = 2
_NS = 16
_NW = _NC * _NS
_C = 1024
_NB = _C // 128
_NCHUNKS = N_EDGES // _C
_ROUNDS = -(-_NCHUNKS // _NW)
_G = _C // 16
_SUB = 128
_NSUB = _C // _SUB

_P0 = 0.99999528
_P1 = -0.49993091
_P2 = 0.04151172
_P3 = -0.00127871
_HALF_SCALE = 0.3141592653589793
_HMAX = 1.5707963267948966


def _sc_body(coords, src_h, dst_h, vec_o, d_o, sw_o,
             idx_s0, idx_d0, rows_s0, rows_d0,
             idx_s1, idx_d1, rows_s1, rows_d1,
             vecb, db, swb,
             sem_a0, sem_b0, sem_a1, sem_b1):
    wid = lax.axis_index("s") * _NC + lax.axis_index("c")
    iota = lax.iota(jnp.int32, 16)
    col0 = jnp.zeros((16,), jnp.int32)
    col1 = col0 + 1
    col2 = col0 + 2
    zero16 = jnp.zeros((16,), jnp.float32)

    bufs = (
        (idx_s0, idx_d0, rows_s0, rows_d0, sem_a0, sem_b0),
        (idx_s1, idx_d1, rows_s1, rows_d1, sem_a1, sem_b1),
    )

    def zinit(k, carry):
        vecb[k // 8, 3, pl.ds((k % 8) * 16, 16)] = zero16
        return carry

    lax.fori_loop(0, _NB * 8, zinit, 0)

    def fetch(par, c):
        idx_s, idx_d, rows_s, rows_d, sem_a, sem_b = bufs[par]
        base = c * _C
        pltpu.sync_copy(src_h.at[pl.ds(base, _C)], idx_s)
        pltpu.sync_copy(dst_h.at[pl.ds(base, _C)], idx_d)
        for j in range(_NSUB):
            sl = pl.ds(j * _SUB, _SUB)
            pltpu.async_copy(coords.at[idx_s.at[sl]], rows_s.at[sl], sem_a)
            pltpu.async_copy(coords.at[idx_d.at[sl]], rows_d.at[sl], sem_b)

    def drain(par):
        idx_s, idx_d, rows_s, rows_d, sem_a, sem_b = bufs[par]
        hbm_dummy = coords.at[pl.ds(0, _SUB)]
        for j in range(_NSUB):
            sl = pl.ds(j * _SUB, _SUB)
            pltpu.make_async_copy(hbm_dummy, rows_s.at[sl], sem_a).wait()
            pltpu.make_async_copy(hbm_dummy, rows_d.at[sl], sem_b).wait()

    def compute(par, c):
        idx_s, idx_d, rows_s, rows_d, sem_a, sem_b = bufs[par]
        base = c * _C

        def group(g, carry2):
            r = g * 16 + iota
            blk = g // 8
            off = (g % 8) * 16
            dx = plsc.load_gather(rows_d, [r, col0]) - plsc.load_gather(rows_s, [r, col0])
            dy = plsc.load_gather(rows_d, [r, col1]) - plsc.load_gather(rows_s, [r, col1])
            dz = plsc.load_gather(rows_d, [r, col2]) - plsc.load_gather(rows_s, [r, col2])
            vecb[blk, 0, pl.ds(off, 16)] = dx
            vecb[blk, 1, pl.ds(off, 16)] = dy
            vecb[blk, 2, pl.ds(off, 16)] = dz
            d2 = dx * dx + dy * dy + dz * dz
            bits = plsc.bitcast(d2, jnp.int32)
            y = plsc.bitcast(jnp.int32(0x5F3759DF) - (bits >> 1), jnp.float32)
            y = y * (1.5 - (0.5 * d2 * y) * y)
            y = y * (1.5 - (0.5 * d2 * y) * y)
            y = y * (1.5 - (0.5 * d2 * y) * y)
            d = d2 * y
            d = jnp.where(d2 == 0.0, 0.0, d)
            db[pl.ds(g * 16, 16)] = d
            h = jnp.minimum(d * _HALF_SCALE, _HMAX)
            u = h * h
            p = _P0 + u * (_P1 + u * (_P2 + u * _P3))
            swb[pl.ds(g * 16, 16)] = jnp.where(d < CUTOFF, p * p, 0.0)
            return carry2

        lax.fori_loop(0, _G, group, 0)
        pltpu.sync_copy(vecb, vec_o.at[pl.ds(c * _NB, _NB)])
        pltpu.sync_copy(db, d_o.at[pl.ds(base, _C)])
        pltpu.sync_copy(swb, sw_o.at[pl.ds(base, _C)])

    @pl.when(wid < _NCHUNKS)
    def _():
        fetch(0, wid)

    def pair(k2, carry):
        for par in (0, 1):
            k = 2 * k2 + par
            c = wid + k * _NW
            cn = c + _NW

            @pl.when(cn < _NCHUNKS)
            def _():
                fetch(1 - par, cn)

            @pl.when(c < _NCHUNKS)
            def _():
                drain(par)
                compute(par, c)
        return carry

    lax.fori_loop(0, _ROUNDS // 2, pair, 0)


_sc_call = pl.kernel(
    _sc_body,
    out_type=[
        jax.ShapeDtypeStruct((N_EDGES // 128, 4, 128), jnp.float32),
        jax.ShapeDtypeStruct((N_EDGES,), jnp.float32),
        jax.ShapeDtypeStruct((N_EDGES,), jnp.float32),
    ],
    mesh=plsc.VectorSubcoreMesh(core_axis_name="c", subcore_axis_name="s"),
    compiler_params=pltpu.CompilerParams(use_tc_tiling_on_sc=False,
                                         needs_layout_passes=False),
    scratch_types=[
        pltpu.VMEM((_C,), jnp.int32),
        pltpu.VMEM((_C,), jnp.int32),
        pltpu.VMEM((_C, _D), jnp.float32),
        pltpu.VMEM((_C, _D), jnp.float32),
        pltpu.VMEM((_C,), jnp.int32),
        pltpu.VMEM((_C,), jnp.int32),
        pltpu.VMEM((_C, _D), jnp.float32),
        pltpu.VMEM((_C, _D), jnp.float32),
        pltpu.VMEM((_NB, 4, 128), jnp.float32),
        pltpu.VMEM((_C,), jnp.float32),
        pltpu.VMEM((_C,), jnp.float32),
        pltpu.SemaphoreType.DMA,
        pltpu.SemaphoreType.DMA,
        pltpu.SemaphoreType.DMA,
        pltpu.SemaphoreType.DMA,
    ],
)


@jax.jit
def kernel(coordinates, edge_src, edge_dst):
    table = jnp.pad(coordinates, ((0, 0), (0, _D - 3)))
    vec4, distances, switch = _sc_call(table, edge_src, edge_dst)
    vec = vec4.transpose(0, 2, 1).reshape(N_EDGES, 4)[:, :3]
    edge_mask = distances < CUTOFF
    return vec, distances, switch, edge_mask


# speedup vs baseline: 53.0434x; 1.1647x over previous
import jax
import jax.numpy as jnp
from jax import lax
from jax.experimental import pallas as pl
from jax.experimental.pallas import tpu as pltpu
from jax.experimental.pallas import tpu_sc as plsc

N_EDGES = 6_400_000
CUTOFF = 5.0

_D = 8
_NC = 2
_NS = 16
_NW = _NC * _NS
_C = 1024
_NB = _C // 128
_NCHUNKS = N_EDGES // _C
_ROUNDS = -(-_NCHUNKS // _NW)
_G = _C // 16
_SUB = 128
_NSUB = _C // _SUB

_P0 = 0.99999528
_P1 = -0.49993091
_P2 = 0.04151172
_P3 = -0.00127871
_HALF_SCALE = 0.3141592653589793
_HMAX = 1.5707963267948966


def _sc_body(coords, src_h, dst_h, vec_o, d_o, sw_o,
             idx_s0, idx_d0, rows_s0, rows_d0,
             idx_s1, idx_d1, rows_s1, rows_d1,
             vecb, db, swb,
             sem_a0, sem_b0, sem_a1, sem_b1):
    wid = lax.axis_index("s") * _NC + lax.axis_index("c")
    iota = lax.iota(jnp.int32, 16)
    col0 = jnp.zeros((16,), jnp.int32)
    col1 = col0 + 1
    col2 = col0 + 2
    zero16 = jnp.zeros((16,), jnp.float32)

    bufs = (
        (idx_s0, idx_d0, rows_s0, rows_d0, sem_a0, sem_b0),
        (idx_s1, idx_d1, rows_s1, rows_d1, sem_a1, sem_b1),
    )

    def zinit(k, carry):
        vecb[k // 8, 3, pl.ds((k % 8) * 16, 16)] = zero16
        return carry

    lax.fori_loop(0, _NB * 8, zinit, 0)

    def fetch(par, c):
        idx_s, idx_d, rows_s, rows_d, sem_a, sem_b = bufs[par]
        base = c * _C
        pltpu.sync_copy(src_h.at[pl.ds(base, _C)], idx_s)
        pltpu.sync_copy(dst_h.at[pl.ds(base, _C)], idx_d)
        for j in range(_NSUB):
            sl = pl.ds(j * _SUB, _SUB)
            pltpu.async_copy(coords.at[idx_s.at[sl]], rows_s.at[sl], sem_a)
            pltpu.async_copy(coords.at[idx_d.at[sl]], rows_d.at[sl], sem_b)

    def drain(par):
        idx_s, idx_d, rows_s, rows_d, sem_a, sem_b = bufs[par]
        hbm_dummy = coords.at[pl.ds(0, _SUB)]
        for j in range(_NSUB):
            sl = pl.ds(j * _SUB, _SUB)
            pltpu.make_async_copy(hbm_dummy, rows_s.at[sl], sem_a).wait()
            pltpu.make_async_copy(hbm_dummy, rows_d.at[sl], sem_b).wait()

    def compute(par, c):
        idx_s, idx_d, rows_s, rows_d, sem_a, sem_b = bufs[par]
        base = c * _C

        def group(g, carry2):
            r = g * 16 + iota
            blk = g // 8
            off = (g % 8) * 16
            dx = plsc.load_gather(rows_d, [r, col0]) - plsc.load_gather(rows_s, [r, col0])
            dy = plsc.load_gather(rows_d, [r, col1]) - plsc.load_gather(rows_s, [r, col1])
            dz = plsc.load_gather(rows_d, [r, col2]) - plsc.load_gather(rows_s, [r, col2])
            vecb[blk, 0, pl.ds(off, 16)] = dx
            vecb[blk, 1, pl.ds(off, 16)] = dy
            vecb[blk, 2, pl.ds(off, 16)] = dz
            d2 = dx * dx + dy * dy + dz * dz
            bits = plsc.bitcast(d2, jnp.int32)
            y = plsc.bitcast(jnp.int32(0x5F3759DF) - (bits >> 1), jnp.float32)
            y = y * (1.5 - (0.5 * d2 * y) * y)
            y = y * (1.5 - (0.5 * d2 * y) * y)
            y = y * (1.5 - (0.5 * d2 * y) * y)
            d = d2 * y
            d = jnp.where(d2 == 0.0, 0.0, d)
            db[pl.ds(g * 16, 16)] = d
            h = jnp.minimum(d * _HALF_SCALE, _HMAX)
            u = h * h
            p = _P0 + u * (_P1 + u * (_P2 + u * _P3))
            swb[pl.ds(g * 16, 16)] = jnp.where(d < CUTOFF, p * p, 0.0)
            return carry2

        lax.fori_loop(0, _G, group, 0)
        pltpu.sync_copy(vecb, vec_o.at[pl.ds(c * _NB, _NB)])
        pltpu.sync_copy(db, d_o.at[pl.ds(base, _C)])
        pltpu.sync_copy(swb, sw_o.at[pl.ds(base, _C)])

    @pl.when(wid < _NCHUNKS)
    def _():
        fetch(0, wid)

    def pair(k2, carry):
        for par in (0, 1):
            k = 2 * k2 + par
            c = wid + k * _NW
            cn = c + _NW

            @pl.when(cn < _NCHUNKS)
            def _():
                fetch(1 - par, cn)

            @pl.when(c < _NCHUNKS)
            def _():
                drain(par)
                compute(par, c)
        return carry

    lax.fori_loop(0, _ROUNDS // 2, pair, 0)


_sc_call = pl.kernel(
    _sc_body,
    out_type=[
        jax.ShapeDtypeStruct((N_EDGES // 128, 4, 128), jnp.float32),
        jax.ShapeDtypeStruct((N_EDGES,), jnp.float32),
        jax.ShapeDtypeStruct((N_EDGES,), jnp.float32),
    ],
    mesh=plsc.VectorSubcoreMesh(core_axis_name="c", subcore_axis_name="s"),
    compiler_params=pltpu.CompilerParams(use_tc_tiling_on_sc=False,
                                         needs_layout_passes=False),
    scratch_types=[
        pltpu.VMEM((_C,), jnp.int32),
        pltpu.VMEM((_C,), jnp.int32),
        pltpu.VMEM((_C, _D), jnp.float32),
        pltpu.VMEM((_C, _D), jnp.float32),
        pltpu.VMEM((_C,), jnp.int32),
        pltpu.VMEM((_C,), jnp.int32),
        pltpu.VMEM((_C, _D), jnp.float32),
        pltpu.VMEM((_C, _D), jnp.float32),
        pltpu.VMEM((_NB, 4, 128), jnp.float32),
        pltpu.VMEM((_C,), jnp.float32),
        pltpu.VMEM((_C,), jnp.float32),
        pltpu.SemaphoreType.DMA,
        pltpu.SemaphoreType.DMA,
        pltpu.SemaphoreType.DMA,
        pltpu.SemaphoreType.DMA,
    ],
)


@jax.jit
def kernel(coordinates, edge_src, edge_dst):
    table = jnp.pad(coordinates, ((0, 0), (0, _D - 3)))
    vec4, distances, switch = _sc_call(table, edge_src, edge_dst)
    vec = vec4.transpose(0, 2, 1).reshape(N_EDGES, 4)[:, :3]
    edge_mask = distances < CUTOFF
    return vec, distances, switch, edge_mask


# speedup vs baseline: 60.4189x; 1.1390x over previous
import jax
import jax.numpy as jnp
from jax import lax
from jax.experimental import pallas as pl
from jax.experimental.pallas import tpu as pltpu
from jax.experimental.pallas import tpu_sc as plsc

N_EDGES = 6_400_000
CUTOFF = 5.0

_D = 8
_NC = 2
_NS = 16
_NW = _NC * _NS
_C = 2048
_NB = _C // 128
_NCHUNKS = N_EDGES // _C
_ROUNDS = -(-_NCHUNKS // _NW)
_G = _C // 16
_SUB = 128
_NSUB = _C // _SUB

_P0 = 0.99999528
_P1 = -0.49993091
_P2 = 0.04151172
_P3 = -0.00127871
_HALF_SCALE = 0.3141592653589793
_HMAX = 1.5707963267948966


def _sc_body(coords, src_h, dst_h, vec_o, d_o, sw_o,
             idx_s0, idx_d0, rows_s0, rows_d0,
             idx_s1, idx_d1, rows_s1, rows_d1,
             vecb, db, swb,
             sem_a0, sem_b0, sem_a1, sem_b1):
    wid = lax.axis_index("s") * _NC + lax.axis_index("c")
    iota = lax.iota(jnp.int32, 16)
    col0 = jnp.zeros((16,), jnp.int32)
    col1 = col0 + 1
    col2 = col0 + 2
    zero16 = jnp.zeros((16,), jnp.float32)

    bufs = (
        (idx_s0, idx_d0, rows_s0, rows_d0, sem_a0, sem_b0),
        (idx_s1, idx_d1, rows_s1, rows_d1, sem_a1, sem_b1),
    )

    def zinit(k, carry):
        vecb[k // 8, 3, pl.ds((k % 8) * 16, 16)] = zero16
        return carry

    lax.fori_loop(0, _NB * 8, zinit, 0)

    def fetch(par, c):
        idx_s, idx_d, rows_s, rows_d, sem_a, sem_b = bufs[par]
        base = c * _C
        pltpu.sync_copy(src_h.at[pl.ds(base, _C)], idx_s)
        pltpu.sync_copy(dst_h.at[pl.ds(base, _C)], idx_d)
        for j in range(_NSUB):
            sl = pl.ds(j * _SUB, _SUB)
            pltpu.async_copy(coords.at[idx_s.at[sl]], rows_s.at[sl], sem_a)
            pltpu.async_copy(coords.at[idx_d.at[sl]], rows_d.at[sl], sem_b)

    def drain(par):
        idx_s, idx_d, rows_s, rows_d, sem_a, sem_b = bufs[par]
        hbm_dummy = coords.at[pl.ds(0, _SUB)]
        for j in range(_NSUB):
            sl = pl.ds(j * _SUB, _SUB)
            pltpu.make_async_copy(hbm_dummy, rows_s.at[sl], sem_a).wait()
            pltpu.make_async_copy(hbm_dummy, rows_d.at[sl], sem_b).wait()

    def compute(par, c):
        idx_s, idx_d, rows_s, rows_d, sem_a, sem_b = bufs[par]
        base = c * _C

        def group(g, carry2):
            r = g * 16 + iota
            blk = g // 8
            off = (g % 8) * 16
            dx = plsc.load_gather(rows_d, [r, col0]) - plsc.load_gather(rows_s, [r, col0])
            dy = plsc.load_gather(rows_d, [r, col1]) - plsc.load_gather(rows_s, [r, col1])
            dz = plsc.load_gather(rows_d, [r, col2]) - plsc.load_gather(rows_s, [r, col2])
            vecb[blk, 0, pl.ds(off, 16)] = dx
            vecb[blk, 1, pl.ds(off, 16)] = dy
            vecb[blk, 2, pl.ds(off, 16)] = dz
            d2 = dx * dx + dy * dy + dz * dz
            bits = plsc.bitcast(d2, jnp.int32)
            y = plsc.bitcast(jnp.int32(0x5F3759DF) - (bits >> 1), jnp.float32)
            y = y * (1.5 - (0.5 * d2 * y) * y)
            y = y * (1.5 - (0.5 * d2 * y) * y)
            y = y * (1.5 - (0.5 * d2 * y) * y)
            d = d2 * y
            d = jnp.where(d2 == 0.0, 0.0, d)
            db[pl.ds(g * 16, 16)] = d
            h = jnp.minimum(d * _HALF_SCALE, _HMAX)
            u = h * h
            p = _P0 + u * (_P1 + u * (_P2 + u * _P3))
            swb[pl.ds(g * 16, 16)] = jnp.where(d < CUTOFF, p * p, 0.0)
            return carry2

        lax.fori_loop(0, _G, group, 0)
        pltpu.sync_copy(vecb, vec_o.at[pl.ds(c * _NB, _NB)])
        pltpu.sync_copy(db, d_o.at[pl.ds(base, _C)])
        pltpu.sync_copy(swb, sw_o.at[pl.ds(base, _C)])

    @pl.when(wid < _NCHUNKS)
    def _():
        fetch(0, wid)

    def pair(k2, carry):
        for par in (0, 1):
            k = 2 * k2 + par
            c = wid + k * _NW
            cn = c + _NW

            @pl.when(cn < _NCHUNKS)
            def _():
                fetch(1 - par, cn)

            @pl.when(c < _NCHUNKS)
            def _():
                drain(par)
                compute(par, c)
        return carry

    lax.fori_loop(0, _ROUNDS // 2, pair, 0)


_sc_call = pl.kernel(
    _sc_body,
    out_type=[
        jax.ShapeDtypeStruct((N_EDGES // 128, 4, 128), jnp.float32),
        jax.ShapeDtypeStruct((N_EDGES,), jnp.float32),
        jax.ShapeDtypeStruct((N_EDGES,), jnp.float32),
    ],
    mesh=plsc.VectorSubcoreMesh(core_axis_name="c", subcore_axis_name="s"),
    compiler_params=pltpu.CompilerParams(use_tc_tiling_on_sc=False,
                                         needs_layout_passes=False),
    scratch_types=[
        pltpu.VMEM((_C,), jnp.int32),
        pltpu.VMEM((_C,), jnp.int32),
        pltpu.VMEM((_C, _D), jnp.float32),
        pltpu.VMEM((_C, _D), jnp.float32),
        pltpu.VMEM((_C,), jnp.int32),
        pltpu.VMEM((_C,), jnp.int32),
        pltpu.VMEM((_C, _D), jnp.float32),
        pltpu.VMEM((_C, _D), jnp.float32),
        pltpu.VMEM((_NB, 4, 128), jnp.float32),
        pltpu.VMEM((_C,), jnp.float32),
        pltpu.VMEM((_C,), jnp.float32),
        pltpu.SemaphoreType.DMA,
        pltpu.SemaphoreType.DMA,
        pltpu.SemaphoreType.DMA,
        pltpu.SemaphoreType.DMA,
    ],
)


@jax.jit
def kernel(coordinates, edge_src, edge_dst):
    table = jnp.pad(coordinates, ((0, 0), (0, _D - 3)))
    vec4, distances, switch = _sc_call(table, edge_src, edge_dst)
    vec = vec4.transpose(0, 2, 1).reshape(N_EDGES, 4)[:, :3]
    edge_mask = distances < CUTOFF
    return vec, distances, switch, edge_mask


# speedup vs baseline: 63.3330x; 1.0482x over previous
import jax
import jax.numpy as jnp
from jax import lax
from jax.experimental import pallas as pl
from jax.experimental.pallas import tpu as pltpu
from jax.experimental.pallas import tpu_sc as plsc

N_EDGES = 6_400_000
CUTOFF = 5.0

_D = 8
_NC = 2
_NS = 16
_NW = _NC * _NS
_C = 2048
_NB = _C // 128
_NCHUNKS = N_EDGES // _C
_ROUNDS = -(-_NCHUNKS // _NW)
_G = _C // 16
_SUB = 128
_NSUB = _C // _SUB

_P0 = 0.99999528
_P1 = -0.49993091
_P2 = 0.04151172
_P3 = -0.00127871
_HALF_SCALE = 0.3141592653589793
_HMAX = 1.5707963267948966


def _sc_body(coords, src_h, dst_h, vec_o, d_o, sw_o,
             idx_s0, idx_d0, rows_s0, rows_d0,
             idx_s1, idx_d1, rows_s1, rows_d1,
             vecb0, db0, swb0, vecb1, db1, swb1,
             sem_a0, sem_b0, sem_a1, sem_b1, sem_o0, sem_o1):
    wid = lax.axis_index("s") * _NC + lax.axis_index("c")
    iota = lax.iota(jnp.int32, 16)
    col0 = jnp.zeros((16,), jnp.int32)
    col1 = col0 + 1
    col2 = col0 + 2
    zero16 = jnp.zeros((16,), jnp.float32)

    bufs = (
        (idx_s0, idx_d0, rows_s0, rows_d0, sem_a0, sem_b0),
        (idx_s1, idx_d1, rows_s1, rows_d1, sem_a1, sem_b1),
    )
    obufs = ((vecb0, db0, swb0, sem_o0), (vecb1, db1, swb1, sem_o1))

    def zinit(k, carry):
        vecb0[k // 8, 3, pl.ds((k % 8) * 16, 16)] = zero16
        vecb1[k // 8, 3, pl.ds((k % 8) * 16, 16)] = zero16
        return carry

    lax.fori_loop(0, _NB * 8, zinit, 0)

    def fetch(par, c):
        idx_s, idx_d, rows_s, rows_d, sem_a, sem_b = bufs[par]
        base = c * _C
        pltpu.sync_copy(src_h.at[pl.ds(base, _C)], idx_s)
        pltpu.sync_copy(dst_h.at[pl.ds(base, _C)], idx_d)
        for j in range(_NSUB):
            sl = pl.ds(j * _SUB, _SUB)
            pltpu.async_copy(coords.at[idx_s.at[sl]], rows_s.at[sl], sem_a)
            pltpu.async_copy(coords.at[idx_d.at[sl]], rows_d.at[sl], sem_b)

    def drain(par):
        idx_s, idx_d, rows_s, rows_d, sem_a, sem_b = bufs[par]
        hbm_dummy = coords.at[pl.ds(0, _SUB)]
        for j in range(_NSUB):
            sl = pl.ds(j * _SUB, _SUB)
            pltpu.make_async_copy(hbm_dummy, rows_s.at[sl], sem_a).wait()
            pltpu.make_async_copy(hbm_dummy, rows_d.at[sl], sem_b).wait()

    def drain_out(par, c):
        vecb, db, swb, sem_o = obufs[par]
        pltpu.make_async_copy(vecb, vec_o.at[pl.ds(c * _NB, _NB)], sem_o).wait()
        pltpu.make_async_copy(db, d_o.at[pl.ds(c * _C, _C)], sem_o).wait()
        pltpu.make_async_copy(swb, sw_o.at[pl.ds(c * _C, _C)], sem_o).wait()

    def compute(par, c):
        idx_s, idx_d, rows_s, rows_d, sem_a, sem_b = bufs[par]
        vecb, db, swb, sem_o = obufs[par]
        base = c * _C

        def group(g, carry2):
            r = g * 16 + iota
            blk = g // 8
            off = (g % 8) * 16
            dx = plsc.load_gather(rows_d, [r, col0]) - plsc.load_gather(rows_s, [r, col0])
            dy = plsc.load_gather(rows_d, [r, col1]) - plsc.load_gather(rows_s, [r, col1])
            dz = plsc.load_gather(rows_d, [r, col2]) - plsc.load_gather(rows_s, [r, col2])
            vecb[blk, 0, pl.ds(off, 16)] = dx
            vecb[blk, 1, pl.ds(off, 16)] = dy
            vecb[blk, 2, pl.ds(off, 16)] = dz
            d2 = dx * dx + dy * dy + dz * dz
            bits = plsc.bitcast(d2, jnp.int32)
            y = plsc.bitcast(jnp.int32(0x5F3759DF) - (bits >> 1), jnp.float32)
            y = y * (1.5 - (0.5 * d2 * y) * y)
            y = y * (1.5 - (0.5 * d2 * y) * y)
            y = y * (1.5 - (0.5 * d2 * y) * y)
            d = d2 * y
            d = jnp.where(d2 == 0.0, 0.0, d)
            db[pl.ds(g * 16, 16)] = d
            h = jnp.minimum(d * _HALF_SCALE, _HMAX)
            u = h * h
            p = _P0 + u * (_P1 + u * (_P2 + u * _P3))
            swb[pl.ds(g * 16, 16)] = jnp.where(d < CUTOFF, p * p, 0.0)
            return carry2

        lax.fori_loop(0, _G, group, 0)
        pltpu.async_copy(vecb, vec_o.at[pl.ds(c * _NB, _NB)], sem_o)
        pltpu.async_copy(db, d_o.at[pl.ds(base, _C)], sem_o)
        pltpu.async_copy(swb, sw_o.at[pl.ds(base, _C)], sem_o)

    @pl.when(wid < _NCHUNKS)
    def _():
        fetch(0, wid)

    def pair(k2, carry):
        for par in (0, 1):
            k = 2 * k2 + par
            c = wid + k * _NW
            cn = c + _NW

            @pl.when(cn < _NCHUNKS)
            def _():
                fetch(1 - par, cn)

            @pl.when(jnp.logical_and(c < _NCHUNKS, k2 > 0))
            def _():
                drain_out(par, c)

            @pl.when(c < _NCHUNKS)
            def _():
                drain(par)
                compute(par, c)
        return carry

    lax.fori_loop(0, _ROUNDS // 2, pair, 0)
    drain_out(0, 0)
    drain_out(1, 0)


_sc_call = pl.kernel(
    _sc_body,
    out_type=[
        jax.ShapeDtypeStruct((N_EDGES // 128, 4, 128), jnp.float32),
        jax.ShapeDtypeStruct((N_EDGES,), jnp.float32),
        jax.ShapeDtypeStruct((N_EDGES,), jnp.float32),
    ],
    mesh=plsc.VectorSubcoreMesh(core_axis_name="c", subcore_axis_name="s"),
    compiler_params=pltpu.CompilerParams(use_tc_tiling_on_sc=False,
                                         needs_layout_passes=False),
    scratch_types=[
        pltpu.VMEM((_C,), jnp.int32),
        pltpu.VMEM((_C,), jnp.int32),
        pltpu.VMEM((_C, _D), jnp.float32),
        pltpu.VMEM((_C, _D), jnp.float32),
        pltpu.VMEM((_C,), jnp.int32),
        pltpu.VMEM((_C,), jnp.int32),
        pltpu.VMEM((_C, _D), jnp.float32),
        pltpu.VMEM((_C, _D), jnp.float32),
        pltpu.VMEM((_NB, 4, 128), jnp.float32),
        pltpu.VMEM((_C,), jnp.float32),
        pltpu.VMEM((_C,), jnp.float32),
        pltpu.VMEM((_NB, 4, 128), jnp.float32),
        pltpu.VMEM((_C,), jnp.float32),
        pltpu.VMEM((_C,), jnp.float32),
        pltpu.SemaphoreType.DMA,
        pltpu.SemaphoreType.DMA,
        pltpu.SemaphoreType.DMA,
        pltpu.SemaphoreType.DMA,
        pltpu.SemaphoreType.DMA,
        pltpu.SemaphoreType.DMA,
    ],
)


@jax.jit
def kernel(coordinates, edge_src, edge_dst):
    table = jnp.pad(coordinates, ((0, 0), (0, _D - 3)))
    vec4, distances, switch = _sc_call(table, edge_src, edge_dst)
    vec = vec4.transpose(0, 2, 1).reshape(N_EDGES, 4)[:, :3]
    edge_mask = distances < CUTOFF
    return vec, distances, switch, edge_mask
